# baseline (speedup 1.0000x reference)
import functools

import jax
import jax.numpy as jnp
from jax.experimental import pallas as pl
from jax.experimental.pallas import tpu as pltpu

B = 16384
D = 128
CH = 2048
NCH = B // CH


def _pipe_kernel(mc_hbm, m_ref, wg_ref, b_ref, out_hbm,
                 in_buf, out_buf, in_sem, out_sem):
    m = m_ref[...]
    wg = wg_ref[...]
    w1 = wg[:, :D]
    w2 = wg[:, D:]
    v = jax.lax.dot_general(m, w1, (((1,), (1,)), ((), ())),
                            preferred_element_type=jnp.float32) + b_ref[...]

    def in_copy(i, slot):
        return pltpu.make_async_copy(
            mc_hbm.at[pl.ds(i * CH, CH)], in_buf.at[slot], in_sem.at[slot])

    def out_copy(i, slot):
        return pltpu.make_async_copy(
            out_buf.at[slot], out_hbm.at[pl.ds(i * CH, CH)], out_sem.at[slot])

    in_copy(0, 0).start()
    for i in range(NCH):
        slot = i % 2
        if i + 1 < NCH:
            in_copy(i + 1, 1 - slot).start()
        in_copy(i, slot).wait()
        if i >= 2:
            out_copy(i - 2, slot).wait()
        mc = in_buf[slot]
        logits = jax.lax.dot_general(mc, w2, (((1,), (1,)), ((), ())),
                                     preferred_element_type=jnp.float32) + v
        gate = jax.nn.sigmoid(logits)
        out_buf[slot] = gate * (m - mc) + mc
        out_copy(i, slot).start()
    out_copy(NCH - 2, (NCH - 2) % 2).wait()
    out_copy(NCH - 1, (NCH - 1) % 2).wait()


@functools.partial(jax.jit, donate_argnums=())
def kernel(current_state, memory_context, W_attn, b_attn, W_gate, b_gate):
    del W_attn, b_attn
    m = jax.lax.stop_gradient(current_state[0:1])
    b2 = b_gate.reshape(1, D)
    return pl.pallas_call(
        _pipe_kernel,
        in_specs=[
            pl.BlockSpec(memory_space=pl.ANY),
            pl.BlockSpec(memory_space=pltpu.VMEM),
            pl.BlockSpec(memory_space=pltpu.VMEM),
            pl.BlockSpec(memory_space=pltpu.VMEM),
        ],
        out_specs=pl.BlockSpec(memory_space=pl.ANY),
        out_shape=jax.ShapeDtypeStruct((B, D), jnp.float32),
        scratch_shapes=[
            pltpu.VMEM((2, CH, D), jnp.float32),
            pltpu.VMEM((2, CH, D), jnp.float32),
            pltpu.SemaphoreType.DMA((2,)),
            pltpu.SemaphoreType.DMA((2,)),
        ],
    )(memory_context, m, W_gate, b2)


# speedup vs baseline: 1.3834x; 1.3834x over previous
import functools

import jax
import jax.numpy as jnp
from jax.experimental import pallas as pl
from jax.experimental.pallas import tpu as pltpu

B = 16384
D = 128
CH = 2048
NCH = B // CH


def _pipe_kernel(mc_hbm, m_ref, wg_ref, b_ref, out_hbm,
                 in_buf, out_buf, in_sem, out_sem):
    m = m_ref[...]
    wg = wg_ref[...]
    w1 = wg[:, :D]
    w2 = wg[:, D:]
    v = jax.lax.dot_general(m, w1, (((1,), (1,)), ((), ())),
                            preferred_element_type=jnp.float32) + b_ref[...]

    def in_copy(i):
        sl = pl.ds(i * CH, CH)
        return pltpu.make_async_copy(mc_hbm.at[sl], in_buf.at[sl], in_sem.at[i])

    def out_copy(i):
        sl = pl.ds(i * CH, CH)
        return pltpu.make_async_copy(out_buf.at[sl], out_hbm.at[sl], out_sem.at[i])

    for i in range(NCH):
        in_copy(i).start()
    for i in range(NCH):
        in_copy(i).wait()
        sl = pl.ds(i * CH, CH)
        mc = in_buf[sl]
        logits = jax.lax.dot_general(mc, w2, (((1,), (1,)), ((), ())),
                                     preferred_element_type=jnp.float32) + v
        gate = jax.nn.sigmoid(logits)
        out_buf[sl] = gate * (m - mc) + mc
        out_copy(i).start()
    for i in range(NCH):
        out_copy(i).wait()


@functools.partial(jax.jit, donate_argnums=())
def kernel(current_state, memory_context, W_attn, b_attn, W_gate, b_gate):
    del W_attn, b_attn
    m = jax.lax.stop_gradient(current_state[0:1])
    b2 = b_gate.reshape(1, D)
    return pl.pallas_call(
        _pipe_kernel,
        in_specs=[
            pl.BlockSpec(memory_space=pl.ANY),
            pl.BlockSpec(memory_space=pltpu.VMEM),
            pl.BlockSpec(memory_space=pltpu.VMEM),
            pl.BlockSpec(memory_space=pltpu.VMEM),
        ],
        out_specs=pl.BlockSpec(memory_space=pl.ANY),
        out_shape=jax.ShapeDtypeStruct((B, D), jnp.float32),
        scratch_shapes=[
            pltpu.VMEM((B, D), jnp.float32),
            pltpu.VMEM((B, D), jnp.float32),
            pltpu.SemaphoreType.DMA((NCH,)),
            pltpu.SemaphoreType.DMA((NCH,)),
        ],
    )(memory_context, m, W_gate, b2)
